# baseline (speedup 1.0000x reference)
import functools

import jax
import jax.numpy as jnp
from jax import lax
from jax.experimental import pallas as pl
from jax.experimental.pallas import tpu as pltpu
from jax.experimental.pallas import tpu_sc as plsc

N = 5000
NPAD = 5120
B = 512
NB = NPAD // B
TOP = 1000
ROWPAD = 1024
TH = 0.7


def _sup_block(px1, py1, px2, py2, pa, cx1, cy1, cx2, cy2, ca):
    xx1 = jnp.maximum(px1, cx1)
    yy1 = jnp.maximum(py1, cy1)
    xx2 = jnp.minimum(px2, cx2)
    yy2 = jnp.minimum(py2, cy2)
    inter = jnp.maximum(xx2 - xx1, 0.0) * jnp.maximum(yy2 - yy1, 0.0)
    denom = pa + ca - inter + 1e-9
    return (inter > TH * denom).astype(jnp.float32)


def _row0(v):
    rmask = (jax.lax.broadcasted_iota(jnp.int32, (8, B), 0) == 0)
    return jnp.broadcast_to(v, (8, B)) * rmask.astype(jnp.float32)


def _nms_body(x1r, y1r, x2r, y2r, cols5, rois_ref, slot_ref,
              keep_ref, rs_ref):
    slot_ref[...] = jnp.full((NB, 1, B), 1e9, jnp.float32)
    keep_ref[...] = jnp.zeros((NB, 1, B), jnp.float32)
    lane = jax.lax.broadcasted_iota(jnp.int32, (1, B), 1)
    tri = (jax.lax.broadcasted_iota(jnp.int32, (B, B), 0)
           < jax.lax.broadcasted_iota(jnp.int32, (B, B), 1)).astype(jnp.float32)

    def row(ref, j):
        return ref[pl.ds(j, 1), 0, :]

    def colblk(p):
        base = pl.ds(pl.multiple_of(p * B, B), B)
        px1 = cols5[base, 1:2]
        py1 = cols5[base, 2:3]
        px2 = cols5[base, 3:4]
        py2 = cols5[base, 4:5]
        return px1, py1, px2, py2, (px2 - px1) * (py2 - py1)

    def mm(k_row, s):
        out = jax.lax.dot_general(_row0(k_row), s, (((1,), (0,)), ((), ())),
                                  preferred_element_type=jnp.float32)
        return out[0:1, :]

    def blk_body(state):
        j, kept = state
        cx1, cy1, cx2, cy2 = (row(x1r, j), row(y1r, j), row(x2r, j),
                              row(y2r, j))
        car = (cx2 - cx1) * (cy2 - cy1)

        def pbody(p, acc):
            s = _sup_block(*colblk(p), cx1, cy1, cx2, cy2, car)
            kprev = keep_ref[pl.ds(p, 1), 0, :]
            return acc + mm(kprev, s)

        acc = jax.lax.fori_loop(0, j, pbody, jnp.zeros((1, B), jnp.float32))
        valid = (j * B + lane) < N
        incoming = jnp.where((acc == 0.0) & valid, 1.0, 0.0)

        scc = _sup_block(*colblk(j), cx1, cy1, cx2, cy2, car) * tri

        def fcond(s):
            return s[1]

        def fbody(s):
            k, _ = s
            hit = mm(k, scc)
            new = jnp.where(hit == 0.0, incoming, 0.0)
            return new, jnp.any(new != k)

        keep_j, _ = jax.lax.while_loop(fcond, fbody,
                                       (incoming, jnp.array(True)))
        keep_ref[pl.ds(j, 1), 0, :] = keep_j
        return j + 1, kept + jnp.sum(keep_j)

    def blk_cond(state):
        j, kept = state
        return (j < NB) & (kept < float(TOP))

    jstar, _ = jax.lax.while_loop(blk_cond, blk_body,
                                  (jnp.int32(0), jnp.float32(0.0)))

    def rank_body(j, carry):
        bk, bsup = carry
        kr = keep_ref[pl.ds(j, 1), 0, :]
        validr = ((j * B + lane) < N).astype(jnp.float32)
        nkr = (1.0 - kr) * validr
        slot_ref[pl.ds(j, 1), 0, :] = mm(kr, tri) + bk
        rs_ref[pl.ds(j, 1), 0, :] = mm(nkr, tri) + bsup
        return bk + jnp.sum(kr), bsup + jnp.sum(nkr)

    kept_total, _ = jax.lax.fori_loop(
        0, jstar, rank_body, (jnp.float32(0.0), jnp.float32(0.0)))

    def slot_body(j, pmax):
        kr = keep_ref[pl.ds(j, 1), 0, :]
        gidx = j * B + lane
        validr = gidx < N
        s = jnp.where(kr > 0.0, slot_ref[pl.ds(j, 1), 0, :],
                      kept_total + rs_ref[pl.ds(j, 1), 0, :])
        s = jnp.where(validr, s, 1e9)
        slot_ref[pl.ds(j, 1), 0, :] = s
        live = (s < float(TOP))
        return jnp.maximum(
            pmax, jnp.max(jnp.where(live, gidx.astype(jnp.float32), -1.0)))

    pmax = jax.lax.fori_loop(0, jstar, slot_body, jnp.float32(0.0))

    nblk = (pmax.astype(jnp.int32) // B) + 1
    for t in range(ROWPAD // 128):
        srow = (jax.lax.broadcasted_iota(jnp.int32, (128, 1), 0)
                + t * 128).astype(jnp.float32)

        def sel_body(jj, acc, srow=srow):
            slot_row = slot_ref[pl.ds(jj, 1), 0, :]
            a = (slot_row == srow).astype(jnp.float32)
            r = cols5[pl.ds(pl.multiple_of(jj * B, B), B), :]
            return acc + jax.lax.dot_general(
                a, r, (((1,), (0,)), ((), ())),
                preferred_element_type=jnp.float32)

        acc = jax.lax.fori_loop(0, nblk, sel_body,
                                jnp.zeros((128, 5), jnp.float32))
        rois_ref[t * 128:(t + 1) * 128, :] = acc


@jax.jit
def _nms_rois(ss, x1, y1, x2, y2):
    rows = [v.reshape(NB, 1, B) for v in (x1, y1, x2, y2)]
    cols5 = jnp.stack((ss, x1, y1, x2, y2), axis=1)
    rois = pl.pallas_call(
        _nms_body,
        out_shape=jax.ShapeDtypeStruct((ROWPAD, 5), jnp.float32),
        scratch_shapes=[pltpu.VMEM((NB, 1, B), jnp.float32),
                        pltpu.VMEM((NB, 1, B), jnp.float32),
                        pltpu.VMEM((NB, 1, B), jnp.float32)],
    )(*rows, cols5)
    return rois


def kernel(boxes, scores, post_nms_top_n):
    srt = lax.sort(
        (-scores, boxes[:, 0], boxes[:, 1], boxes[:, 2], boxes[:, 3]),
        num_keys=1, is_stable=True)
    pad = lambda v: jnp.pad(v, (0, NPAD - N))
    sneg, sx1, sy1, sx2, sy2 = (pad(v) for v in srt)
    return _nms_rois(-sneg, sx1, sy1, sx2, sy2)[:TOP]
